# baseline (speedup 1.0000x reference)
import functools

import jax
import jax.numpy as jnp
from jax import lax
from jax.experimental import pallas as pl
from jax.experimental.pallas import tpu as pltpu
from jax.experimental.pallas import tpu_sc as plsc

B = 32768
K = 8192
D = 32
BB = 2048
KC = 512
NB = B // BB
NKC = K // KC
LW = 128
RG = 256
COMMITMENT = 0.25


def _argmin_body(x_ref, cb_ref, idx_ref, loss_ref, bsc_ref):
    i = pl.program_id(0)
    x = x_ref[...]
    a = jnp.sum(x * x, axis=1, keepdims=True)
    x2 = x * (-2.0)

    @pl.when(i == 0)
    def _():
        cb = cb_ref[...]
        bsc_ref[...] = jnp.sum(cb * cb, axis=1)

    @pl.when(i == 0)
    def _():
        loss_ref[...] = jnp.zeros((1, 1), dtype=jnp.float32)

    NR = BB // RG
    lsum = jnp.zeros((1, 1), dtype=jnp.float32)
    for r in range(NR):
        xr = x2[r * RG:(r + 1) * RG]
        ar = a[r * RG:(r + 1) * RG]
        bv = jnp.full((RG, LW), jnp.inf, dtype=jnp.float32)
        bi = jnp.zeros((RG, LW), dtype=jnp.int32)
        for k in range(NKC):
            cbk = cb_ref[pl.ds(k * KC, KC), :]
            bk = bsc_ref[pl.ds(k * KC, KC)]
            m2 = lax.dot_general(xr, cbk, (((1,), (1,)), ((), ())),
                                 preferred_element_type=jnp.float32)
            d = (ar + bk[None, :]) + m2
            d0, d1 = d[:, 0:128], d[:, 128:256]
            d2, d3 = d[:, 256:384], d[:, 384:512]
            m01 = jnp.minimum(d0, d1)
            g01 = jnp.where(d1 < d0, jnp.int32(128), jnp.int32(0))
            m23 = jnp.minimum(d2, d3)
            g23 = jnp.where(d3 < d2, jnp.int32(384), jnp.int32(256))
            dmin = jnp.minimum(m01, m23)
            gbase = jnp.where(m23 < m01, g23 + k * KC, g01 + k * KC)
            upd = dmin < bv
            bv = jnp.minimum(bv, dmin)
            bi = jnp.where(upd, gbase, bi)

        full_idx = bi + lax.broadcasted_iota(jnp.int32, (RG, LW), 1)
        minv = jnp.min(bv, axis=1, keepdims=True)
        idxm = jnp.where(bv == minv, full_idx, jnp.int32(2**31 - 1))
        idx_ref[pl.ds(r * RG, RG)] = jnp.min(idxm, axis=1)
        lsum = lsum + jnp.sum(minv).reshape(1, 1)

    loss_ref[...] += lsum


_dist_argmin = pl.pallas_call(
    _argmin_body,
    grid=(NB,),
    in_specs=[
        pl.BlockSpec((BB, D), lambda i: (i, 0)),
        pl.BlockSpec((K, D), lambda i: (0, 0)),
    ],
    out_specs=[
        pl.BlockSpec((BB,), lambda i: (i,)),
        pl.BlockSpec((1, 1), lambda i: (0, 0)),
    ],
    out_shape=[
        jax.ShapeDtypeStruct((B,), jnp.int32),
        jax.ShapeDtypeStruct((1, 1), jnp.float32),
    ],
    scratch_shapes=[pltpu.VMEM((K,), jnp.float32)],
)


_NW = 32
_NCORES = 2
_BPW = B // _NW
_CH = 128
_NCH = _BPW // _CH


@functools.cache
def _make_gather():
    mesh = plsc.VectorSubcoreMesh(core_axis_name="c", subcore_axis_name="s")

    @functools.partial(
        pl.kernel,
        mesh=mesh,
        out_type=jax.ShapeDtypeStruct((_NW, _NCH, _CH, D), jnp.float32),
        scratch_types=[
            pltpu.VMEM((_NCH, _CH), jnp.int32),
            pltpu.VMEM((_NCH, _CH, D), jnp.float32),
            pltpu.SemaphoreType.DMA,
        ],
        compiler_params=pltpu.CompilerParams(use_tc_tiling_on_sc=False),
    )
    def _gather_body(cb_hbm, idx_hbm, out_hbm, idx_v, rows_v, sem):
        wid = lax.axis_index("s") * _NCORES + lax.axis_index("c")
        pltpu.sync_copy(idx_hbm.at[wid], idx_v)
        copies = [
            pltpu.async_copy(cb_hbm.at[idx_v.at[j]], rows_v.at[j], sem)
            for j in range(_NCH)
        ]
        for cp in copies:
            cp.wait()
        pltpu.sync_copy(rows_v, out_hbm.at[wid])

    return _gather_body


def kernel(inputs, codebook):
    idx, loss_acc = _dist_argmin(inputs, codebook)
    rows = _make_gather()(codebook, idx.reshape(_NW, _NCH, _CH))
    quantized = rows.reshape(B, D)
    mean_sq = loss_acc[0, 0] / (B * D)
    loss = mean_sq + COMMITMENT * mean_sq
    quantized_st = inputs + (quantized - inputs)
    return quantized_st, loss


# speedup vs baseline: 1.1236x; 1.1236x over previous
import functools

import jax
import jax.numpy as jnp
from jax import lax
from jax.experimental import pallas as pl
from jax.experimental.pallas import tpu as pltpu
from jax.experimental.pallas import tpu_sc as plsc

B = 32768
K = 8192
D = 32
BB = 2048
KC = 512
NB = B // BB
NKC = K // KC
LW = 128
RG = 512
COMMITMENT = 0.25


def _argmin_body(x_ref, cb_ref, idx_ref, loss_ref, bsc_ref):
    i = pl.program_id(0)
    x = x_ref[...]
    a = jnp.sum(x * x, axis=1, keepdims=True)
    x2 = x * (-2.0)

    @pl.when(i == 0)
    def _():
        cb = cb_ref[...]
        bsc_ref[...] = jnp.sum(cb * cb, axis=1)

    NR = BB // RG
    bvs = [jnp.full((RG, LW), jnp.inf, dtype=jnp.float32) for _ in range(NR)]
    bis = [jnp.zeros((RG, LW), dtype=jnp.int32) for _ in range(NR)]
    for k in range(NKC):
        cbk = cb_ref[pl.ds(k * KC, KC), :]
        bk = bsc_ref[pl.ds(k * KC, KC)]
        for r in range(NR):
            xr = x2[r * RG:(r + 1) * RG]
            ar = a[r * RG:(r + 1) * RG]
            m2 = lax.dot_general(xr, cbk, (((1,), (1,)), ((), ())),
                                 preferred_element_type=jnp.float32)
            d = (ar + bk[None, :]) + m2
            d0, d1 = d[:, 0:128], d[:, 128:256]
            d2, d3 = d[:, 256:384], d[:, 384:512]
            m01 = jnp.minimum(d0, d1)
            g01 = jnp.where(d1 < d0, jnp.int32(128), jnp.int32(0))
            m23 = jnp.minimum(d2, d3)
            g23 = jnp.where(d3 < d2, jnp.int32(384), jnp.int32(256))
            dmin = jnp.minimum(m01, m23)
            gbase = jnp.where(m23 < m01, g23 + k * KC, g01 + k * KC)
            upd = dmin < bvs[r]
            bvs[r] = jnp.minimum(bvs[r], dmin)
            bis[r] = jnp.where(upd, gbase, bis[r])

    @pl.when(i == 0)
    def _():
        loss_ref[...] = jnp.zeros((1, 1), dtype=jnp.float32)

    lsum = jnp.zeros((1, 1), dtype=jnp.float32)
    for r in range(NR):
        full_idx = bis[r] + lax.broadcasted_iota(jnp.int32, (RG, LW), 1)
        minv = jnp.min(bvs[r], axis=1, keepdims=True)
        idxm = jnp.where(bvs[r] == minv, full_idx, jnp.int32(2**31 - 1))
        idx_ref[pl.ds(r * RG, RG)] = jnp.min(idxm, axis=1)
        lsum = lsum + jnp.sum(minv).reshape(1, 1)

    loss_ref[...] += lsum


_dist_argmin = pl.pallas_call(
    _argmin_body,
    grid=(NB,),
    in_specs=[
        pl.BlockSpec((BB, D), lambda i: (i, 0)),
        pl.BlockSpec((K, D), lambda i: (0, 0)),
    ],
    out_specs=[
        pl.BlockSpec((BB,), lambda i: (i,)),
        pl.BlockSpec((1, 1), lambda i: (0, 0)),
    ],
    out_shape=[
        jax.ShapeDtypeStruct((B,), jnp.int32),
        jax.ShapeDtypeStruct((1, 1), jnp.float32),
    ],
    scratch_shapes=[pltpu.VMEM((K,), jnp.float32)],
)


_NW = 32
_NCORES = 2
_BPW = B // _NW
_CH = 128
_NCH = _BPW // _CH


@functools.cache
def _make_gather():
    mesh = plsc.VectorSubcoreMesh(core_axis_name="c", subcore_axis_name="s")

    @functools.partial(
        pl.kernel,
        mesh=mesh,
        out_type=jax.ShapeDtypeStruct((_NW, _NCH, _CH, D), jnp.float32),
        scratch_types=[
            pltpu.VMEM((_NCH, _CH), jnp.int32),
            pltpu.VMEM((_NCH, _CH, D), jnp.float32),
            pltpu.SemaphoreType.DMA,
        ],
        compiler_params=pltpu.CompilerParams(use_tc_tiling_on_sc=False),
    )
    def _gather_body(cb_hbm, idx_hbm, out_hbm, idx_v, rows_v, sem):
        wid = lax.axis_index("s") * _NCORES + lax.axis_index("c")
        pltpu.sync_copy(idx_hbm.at[wid], idx_v)
        copies = [
            pltpu.async_copy(cb_hbm.at[idx_v.at[j]], rows_v.at[j], sem)
            for j in range(_NCH)
        ]
        for cp in copies:
            cp.wait()
        pltpu.sync_copy(rows_v, out_hbm.at[wid])

    return _gather_body


def kernel(inputs, codebook):
    idx, loss_acc = _dist_argmin(inputs, codebook)
    rows = _make_gather()(codebook, idx.reshape(_NW, _NCH, _CH))
    quantized = rows.reshape(B, D)
    mean_sq = loss_acc[0, 0] / (B * D)
    loss = mean_sq + COMMITMENT * mean_sq
    quantized_st = inputs + (quantized - inputs)
    return quantized_st, loss
